# baseline (speedup 1.0000x reference)
import functools
import math

import jax
import jax.numpy as jnp
from jax import lax
from jax.experimental import pallas as pl
from jax.experimental.pallas import tpu as pltpu
from jax.experimental.pallas import tpu_sc as plsc

EMBED_DIM = 64
SCALE = math.sqrt(EMBED_DIM)

NUM_CORES = 2
NUM_SUBCORES = 16
NUM_WORKERS = NUM_CORES * NUM_SUBCORES

SROWS = 8
BCOLS = 128
NBUF_IN = 4
NBUF_OUT = 2
LANES = 16


def _build(s_total: int, b_total: int):
    units_s = s_total // SROWS
    units_b = b_total // BCOLS
    units_per_worker = (units_s * units_b) // NUM_WORKERS

    mesh = plsc.VectorSubcoreMesh(core_axis_name="c", subcore_axis_name="s")

    @functools.partial(
        pl.kernel,
        out_type=jax.ShapeDtypeStruct((s_total, EMBED_DIM, b_total),
                                      jnp.float32),
        mesh=mesh,
        scratch_types=(
            [pltpu.VMEM((SROWS, BCOLS), jnp.int32)] * 2
            + [pltpu.VMEM((BCOLS, 2 * EMBED_DIM), jnp.float32)] * NBUF_IN
            + [pltpu.VMEM((EMBED_DIM, BCOLS), jnp.float32)] * NBUF_OUT
            + [pltpu.SemaphoreType.DMA] * (NBUF_IN + NBUF_OUT)
        ),
        compiler_params=pltpu.CompilerParams(needs_layout_passes=False),
    )
    def emb(idsh_hbm, off_hbm, table_hbm, out_hbm, *scratch):
        ids_v, off_v = scratch[0], scratch[1]
        in_bufs = scratch[2:2 + NBUF_IN]
        out_bufs = scratch[2 + NBUF_IN:2 + NBUF_IN + NBUF_OUT]
        sems = scratch[2 + NBUF_IN + NBUF_OUT:]
        sem_in = sems[:NBUF_IN]
        sem_out = sems[NBUF_IN:]

        wid = lax.axis_index("s") * NUM_CORES + lax.axis_index("c")

        def unit_body(t, carry):
            u = wid * units_per_worker + t
            s0 = (u // units_b) * SROWS
            b0 = (u % units_b) * BCOLS

            pltpu.sync_copy(
                idsh_hbm.at[pl.ds(s0, SROWS), pl.ds(b0, BCOLS)], ids_v)
            pltpu.sync_copy(
                off_hbm.at[pl.ds(s0, SROWS), pl.ds(b0, BCOLS)], off_v)

            def fire_gather(si):
                return pltpu.async_copy(
                    table_hbm.at[ids_v.at[si]],
                    in_bufs[si % NBUF_IN],
                    sem_in[si % NBUF_IN])

            gathers = [fire_gather(si) for si in range(NBUF_IN)]
            writebacks = [None] * SROWS

            for si in range(SROWS):
                if si >= NBUF_OUT:
                    writebacks[si - NBUF_OUT].wait()
                gathers[si].wait()

                ib = in_bufs[si % NBUF_IN]
                ob = out_bufs[si % NBUF_OUT]

                def grp_body(grp, carry1):
                    g0 = pl.multiple_of(grp * LANES, LANES)
                    rows16 = lax.iota(jnp.int32, LANES) + grp * LANES
                    cols0 = off_v[si, pl.ds(g0, LANES)]

                    def d_body(d, carry2):
                        vals = plsc.load_gather(ib, [rows16, cols0 + d])
                        ob[d, pl.ds(g0, LANES)] = vals * SCALE
                        return carry2

                    lax.fori_loop(0, EMBED_DIM, d_body, 0, unroll=8)
                    return carry1

                lax.fori_loop(0, BCOLS // LANES, grp_body, 0)

                writebacks[si] = pltpu.async_copy(
                    ob,
                    out_hbm.at[s0 + si, :, pl.ds(b0, BCOLS)],
                    sem_out[si % NBUF_OUT])
                if si + NBUF_IN < SROWS:
                    gathers.append(fire_gather(si + NBUF_IN))

            for si in range(SROWS - NBUF_OUT, SROWS):
                writebacks[si].wait()
            return carry

        lax.fori_loop(0, units_per_worker, unit_body, 0)

    return emb


def kernel(tokens, table):
    b, s = tokens.shape
    vocab, d = table.shape
    tt = jnp.swapaxes(tokens, 0, 1)
    idsh = jax.lax.shift_right_logical(tt, 1)
    off = jnp.left_shift(jnp.bitwise_and(tt, 1), 6)
    tab2 = table.reshape(vocab // 2, 2 * d)
    outp = _build(s, b)(idsh, off, tab2)
    return jnp.transpose(outp, (2, 0, 1))


# speedup vs baseline: 1.4159x; 1.4159x over previous
import functools
import math

import jax
import jax.numpy as jnp
from jax import lax
from jax.experimental import pallas as pl
from jax.experimental.pallas import tpu as pltpu
from jax.experimental.pallas import tpu_sc as plsc

EMBED_DIM = 64
SCALE = math.sqrt(EMBED_DIM)

NUM_CORES = 2
NUM_SUBCORES = 16
NUM_WORKERS = NUM_CORES * NUM_SUBCORES

SROWS = 8
BCOLS = 128
NBUF_IN = 4
NBUF_OUT = 2
LANES = 16


def _build(s_total: int, b_total: int):
    units_s = s_total // SROWS
    units_b = b_total // BCOLS
    units_per_worker = (units_s * units_b) // NUM_WORKERS

    mesh = plsc.VectorSubcoreMesh(core_axis_name="c", subcore_axis_name="s")

    @functools.partial(
        pl.kernel,
        out_type=jax.ShapeDtypeStruct((s_total, EMBED_DIM, b_total),
                                      jnp.float32),
        mesh=mesh,
        scratch_types=(
            [pltpu.VMEM((SROWS, BCOLS), jnp.int32)] * 2
            + [pltpu.VMEM((BCOLS, 2 * EMBED_DIM), jnp.float32)] * NBUF_IN
            + [pltpu.VMEM((EMBED_DIM, BCOLS), jnp.float32)] * NBUF_OUT
            + [pltpu.SemaphoreType.DMA] * (NBUF_IN + NBUF_OUT)
        ),
        compiler_params=pltpu.CompilerParams(needs_layout_passes=False),
    )
    def emb(idsh_hbm, off_hbm, table_hbm, out_hbm, *scratch):
        ids_v, off_v = scratch[0], scratch[1]
        in_bufs = scratch[2:2 + NBUF_IN]
        out_bufs = scratch[2 + NBUF_IN:2 + NBUF_IN + NBUF_OUT]
        sems = scratch[2 + NBUF_IN + NBUF_OUT:]
        sem_in = sems[:NBUF_IN]
        sem_out = sems[NBUF_IN:]

        wid = lax.axis_index("s") * NUM_CORES + lax.axis_index("c")

        def unit_body(t, carry):
            u = wid * units_per_worker + t
            s0 = (u // units_b) * SROWS
            b0 = (u % units_b) * BCOLS

            pltpu.sync_copy(
                idsh_hbm.at[pl.ds(s0, SROWS), pl.ds(b0, BCOLS)], ids_v)
            pltpu.sync_copy(
                off_hbm.at[pl.ds(s0, SROWS), pl.ds(b0, BCOLS)], off_v)

            def fire_gather(si):
                return pltpu.async_copy(
                    table_hbm.at[ids_v.at[si]],
                    in_bufs[si % NBUF_IN],
                    sem_in[si % NBUF_IN])

            gathers = [fire_gather(si) for si in range(NBUF_IN)]
            writebacks = [None] * SROWS

            for si in range(SROWS):
                if si >= NBUF_OUT:
                    writebacks[si - NBUF_OUT].wait()
                gathers[si].wait()

                ib = in_bufs[si % NBUF_IN]
                ob = out_bufs[si % NBUF_OUT]

                def grp_body(grp, carry1):
                    g0 = pl.multiple_of(grp * LANES, LANES)
                    rows16 = lax.iota(jnp.int32, LANES) + grp * LANES
                    cols0 = off_v[si, pl.ds(g0, LANES)]

                    @plsc.parallel_loop(0, EMBED_DIM, 1, unroll=8)
                    def d_body(d):
                        vals = plsc.load_gather(ib, [rows16, cols0 + d])
                        ob[d, pl.ds(g0, LANES)] = vals * SCALE

                    return carry1

                lax.fori_loop(0, BCOLS // LANES, grp_body, 0)

                writebacks[si] = pltpu.async_copy(
                    ob,
                    out_hbm.at[s0 + si, :, pl.ds(b0, BCOLS)],
                    sem_out[si % NBUF_OUT])
                if si + NBUF_IN < SROWS:
                    gathers.append(fire_gather(si + NBUF_IN))

            for si in range(SROWS - NBUF_OUT, SROWS):
                writebacks[si].wait()
            return carry

        lax.fori_loop(0, units_per_worker, unit_body, 0)

    return emb


def kernel(tokens, table):
    b, s = tokens.shape
    vocab, d = table.shape
    tt = jnp.swapaxes(tokens, 0, 1)
    idsh = jax.lax.shift_right_logical(tt, 1)
    off = jnp.left_shift(jnp.bitwise_and(tt, 1), 6)
    tab2 = table.reshape(vocab // 2, 2 * d)
    outp = _build(s, b)(idsh, off, tab2)
    return jnp.transpose(outp, (2, 0, 1))


# speedup vs baseline: 2.4897x; 1.7584x over previous
import functools
import math

import jax
import jax.numpy as jnp
from jax import lax
from jax.experimental import pallas as pl
from jax.experimental.pallas import tpu as pltpu
from jax.experimental.pallas import tpu_sc as plsc

EMBED_DIM = 64
SCALE = math.sqrt(EMBED_DIM)

NUM_CORES = 2
NUM_SUBCORES = 16
NUM_WORKERS = NUM_CORES * NUM_SUBCORES

CHUNK = 128
NBUF = 4
LANES = 16


def _build(total_rows: int):
    rows_per_worker = total_rows // NUM_WORKERS
    chunks_per_worker = rows_per_worker // CHUNK
    rounds = chunks_per_worker // NBUF
    assert rounds * NBUF == chunks_per_worker

    mesh = plsc.VectorSubcoreMesh(core_axis_name="c", subcore_axis_name="s")

    @functools.partial(
        pl.kernel,
        out_type=jax.ShapeDtypeStruct((total_rows, EMBED_DIM), jnp.float32),
        mesh=mesh,
        scratch_types=(
            [pltpu.VMEM((chunks_per_worker, CHUNK), jnp.int32)]
            + [pltpu.VMEM((CHUNK, EMBED_DIM), jnp.float32)] * NBUF
            + [pltpu.SemaphoreType.DMA] * (2 * NBUF)
        ),
        compiler_params=pltpu.CompilerParams(needs_layout_passes=False),
    )
    def emb(tokens_hbm, table_hbm, out_hbm, *scratch):
        ids_all = scratch[0]
        row_bufs = scratch[1:1 + NBUF]
        sem_in = scratch[1 + NBUF:1 + 2 * NBUF]
        sem_out = scratch[1 + 2 * NBUF:]

        wid = lax.axis_index("s") * NUM_CORES + lax.axis_index("c")
        base_row = wid * rows_per_worker

        pltpu.sync_copy(
            tokens_hbm.at[pl.ds(wid * chunks_per_worker, chunks_per_worker)],
            ids_all)

        def round_body(p, carry):
            c0 = p * NBUF

            for b in range(NBUF):
                rb = row_bufs[b]
                for grp in range(CHUNK // LANES):
                    vec = ids_all[c0 + b, pl.ds(grp * LANES, LANES)]
                    for l in range(LANES):
                        pltpu.async_copy(
                            table_hbm.at[vec[l]],
                            rb.at[grp * LANES + l],
                            sem_in[b])

            writebacks = []
            for b in range(NBUF):
                rb = row_bufs[b]
                pltpu.make_async_copy(
                    table_hbm.at[pl.ds(0, CHUNK)], rb, sem_in[b]).wait()

                @plsc.parallel_loop(0, CHUNK, 1, unroll=4)
                def scale_row(i):
                    for j in range(EMBED_DIM // LANES):
                        sl = (i, pl.ds(j * LANES, LANES))
                        rb[sl] = rb[sl] * SCALE

                writebacks.append(pltpu.async_copy(
                    rb,
                    out_hbm.at[pl.ds(base_row + (c0 + b) * CHUNK, CHUNK)],
                    sem_out[b]))
            for wb in writebacks:
                wb.wait()
            return carry

        lax.fori_loop(0, rounds, round_body, 0)

    return emb


def kernel(tokens, table):
    b, s = tokens.shape
    total_rows = b * s
    tokens2d = tokens.reshape(total_rows // CHUNK, CHUNK)
    out = _build(total_rows)(tokens2d, table)
    return out.reshape(b, s, EMBED_DIM)
